# Initial kernel scaffold; baseline (speedup 1.0000x reference)
import jax
import jax.numpy as jnp
from jax.experimental import pallas as pl


def kernel(hidden_states, gate_w, w_gate_up, w_down, ws_gate_up, ws_down):
    raise NotImplementedError("write your pallas kernel here")


# speedup vs baseline: 2.0139x; 2.0139x over previous
import functools

import jax
import jax.numpy as jnp
from jax.experimental import pallas as pl
from jax.experimental.pallas import tpu as pltpu

D = 1024
I = 512
SI = 1024
E = 16
EPAD = 128
NEG = -1e30


def _router_shared_body(x32_ref, xb_ref, gate_ref, ws1_ref, ws2_ref,
                        coeff_ref, shared_ref):
    x32 = x32_ref[...]
    logits = jnp.dot(x32, gate_ref[...], preferred_element_type=jnp.float32)
    lane = jax.lax.broadcasted_iota(jnp.int32, logits.shape, 1)
    valid = lane < E
    logits = jnp.where(valid, logits, NEG)
    m = jnp.max(logits, axis=1, keepdims=True)
    p = jnp.exp(logits - m)
    p = jnp.where(valid, p, 0.0)
    p = p / jnp.sum(p, axis=1, keepdims=True)
    p1 = jnp.max(p, axis=1, keepdims=True)
    i1 = jnp.min(jnp.where(p == p1, lane, EPAD), axis=1, keepdims=True)
    mask1 = lane == i1
    pm = jnp.where(mask1, -1.0, p)
    p2 = jnp.max(pm, axis=1, keepdims=True)
    i2 = jnp.min(jnp.where(pm == p2, lane, EPAD), axis=1, keepdims=True)
    denom = p1 + p2
    coeff = (jnp.where(mask1, p1, 0.0)
             + jnp.where(lane == i2, p2, 0.0)) / denom
    coeff_ref[...] = coeff

    xb = xb_ref[...]
    h = jnp.dot(xb, ws1_ref[...], preferred_element_type=jnp.float32)
    act = (jax.nn.silu(h[:, :SI]) * h[:, SI:]).astype(jnp.bfloat16)
    shared_ref[...] = jnp.dot(act, ws2_ref[...],
                              preferred_element_type=jnp.float32)


def _expert_body(xb_ref, w1_ref, w2_ref, coeff_ref, shared_ref, out_ref):
    e = pl.program_id(1)

    @pl.when(e == 0)
    def _init():
        out_ref[...] = shared_ref[...]

    xb = xb_ref[...]
    h = jnp.dot(xb, w1_ref[0], preferred_element_type=jnp.float32)
    act = (jax.nn.silu(h[:, :I]) * h[:, I:]).astype(jnp.bfloat16)
    y = jnp.dot(act, w2_ref[0], preferred_element_type=jnp.float32)
    lane = jax.lax.broadcasted_iota(jnp.int32, (xb.shape[0], EPAD), 1)
    c = jnp.sum(jnp.where(lane == e, coeff_ref[...], 0.0),
                axis=1, keepdims=True)
    out_ref[...] += y * c


def _moe(x, gate_pad, w1b, w2b, ws1b, ws2b, interpret=False):
    T = x.shape[0]
    xb = x.astype(jnp.bfloat16)

    TMA = 512
    coeff, shared = pl.pallas_call(
        _router_shared_body,
        grid=(T // TMA,),
        in_specs=[
            pl.BlockSpec((TMA, D), lambda t: (t, 0)),
            pl.BlockSpec((TMA, D), lambda t: (t, 0)),
            pl.BlockSpec((D, EPAD), lambda t: (0, 0)),
            pl.BlockSpec((D, 2 * SI), lambda t: (0, 0)),
            pl.BlockSpec((SI, D), lambda t: (0, 0)),
        ],
        out_specs=[
            pl.BlockSpec((TMA, EPAD), lambda t: (t, 0)),
            pl.BlockSpec((TMA, D), lambda t: (t, 0)),
        ],
        out_shape=[
            jax.ShapeDtypeStruct((T, EPAD), jnp.float32),
            jax.ShapeDtypeStruct((T, D), jnp.float32),
        ],
        compiler_params=pltpu.CompilerParams(
            dimension_semantics=("parallel",)),
        interpret=interpret,
    )(x, xb, gate_pad, ws1b, ws2b)

    TMB = 1024
    out = pl.pallas_call(
        _expert_body,
        grid=(T // TMB, E),
        in_specs=[
            pl.BlockSpec((TMB, D), lambda t, e: (t, 0)),
            pl.BlockSpec((1, D, 2 * I), lambda t, e: (e, 0, 0)),
            pl.BlockSpec((1, I, D), lambda t, e: (e, 0, 0)),
            pl.BlockSpec((TMB, EPAD), lambda t, e: (t, 0)),
            pl.BlockSpec((TMB, D), lambda t, e: (t, 0)),
        ],
        out_specs=pl.BlockSpec((TMB, D), lambda t, e: (t, 0)),
        out_shape=jax.ShapeDtypeStruct((T, D), jnp.float32),
        compiler_params=pltpu.CompilerParams(
            dimension_semantics=("parallel", "arbitrary")),
        interpret=interpret,
    )(xb, w1b, w2b, coeff, shared)
    return out


def kernel(hidden_states, gate_w, w_gate_up, w_down, ws_gate_up, ws_down):
    orig_shape = hidden_states.shape
    x = hidden_states.reshape(-1, D)
    gate_pad = jnp.pad(gate_w, ((0, 0), (0, EPAD - E)))
    w1b = w_gate_up.astype(jnp.bfloat16)
    w2b = w_down.astype(jnp.bfloat16)
    ws1b = ws_gate_up.astype(jnp.bfloat16)
    ws2b = ws_down.astype(jnp.bfloat16)
    out = _moe(x, gate_pad, w1b, w2b, ws1b, ws2b)
    return out.reshape(orig_shape)
